# Initial kernel scaffold; baseline (speedup 1.0000x reference)
import jax
import jax.numpy as jnp
from jax.experimental import pallas as pl


def kernel(iw, w_cat, b_cat, lin_w, lin_b, x, edge_index, edge_weight):
    raise NotImplementedError("write your pallas kernel here")


# speedup vs baseline: 1.0041x; 1.0041x over previous
import jax
import jax.numpy as jnp
from jax.experimental import pallas as pl
from jax.experimental.pallas import tpu as pltpu

FP = 128
CP = 128
NCLS = 40
TILE = 256


def _evolve_kernel(iw_ref, wcat_ref, bcat_ref, o_ref):
    h = iw_ref[0]
    fp = h.shape[-1]
    g = jnp.dot(h.astype(jnp.bfloat16), wcat_ref[0],
                preferred_element_type=jnp.float32) + bcat_ref[0]
    r = jax.nn.sigmoid(g[:, :fp] + g[:, 3 * fp:4 * fp])
    z = jax.nn.sigmoid(g[:, fp:2 * fp] + g[:, 4 * fp:5 * fp])
    n = jnp.tanh(g[:, 2 * fp:3 * fp] + r * g[:, 5 * fp:6 * fp])
    o_ref[0] = (1.0 - z) * n + z * h


def _layer1_kernel(a_ref, x_ref, w_ref, o_ref):
    ax = jnp.dot(a_ref[...], x_ref[...], preferred_element_type=jnp.float32)
    h = jnp.maximum(
        jnp.dot(ax.astype(jnp.bfloat16), w_ref[0],
                preferred_element_type=jnp.float32), 0.0)
    o_ref[...] = jnp.dot(h.astype(jnp.bfloat16), w_ref[1],
                         preferred_element_type=jnp.float32).astype(o_ref.dtype)


def _head_kernel(a_ref, y_ref, lw_ref, lb_ref, o_ref):
    h = jnp.maximum(
        jnp.dot(a_ref[...], y_ref[...], preferred_element_type=jnp.float32), 0.0)
    logits = jnp.dot(h.astype(jnp.bfloat16), lw_ref[...],
                     preferred_element_type=jnp.float32) + lb_ref[...]
    m = jnp.max(logits, axis=-1, keepdims=True)
    s = logits - m
    o_ref[...] = s - jnp.log(jnp.sum(jnp.exp(s), axis=-1, keepdims=True))


def _build_adj(edge_index, edge_weight, n):
    src, dst = edge_index[0], edge_index[1]
    has_loop = jnp.zeros((n,), jnp.int32).at[src].max(
        (src == dst).astype(jnp.int32))
    loop_w = jnp.where(has_loop > 0, 0.0, 1.0).astype(edge_weight.dtype)
    deg = loop_w + jnp.zeros((n,), edge_weight.dtype).at[dst].add(edge_weight)
    dinv = jnp.where(deg > 0, jax.lax.rsqrt(deg), 0.0)
    norm = dinv[src] * edge_weight * dinv[dst]
    a = jnp.zeros((n, n), edge_weight.dtype).at[dst, src].add(norm)
    di = jnp.arange(n, dtype=jnp.int32)
    a = a.at[di, di].add(dinv * dinv * loop_w)
    return a.astype(jnp.bfloat16)


def kernel(iw, w_cat, b_cat, lin_w, lin_b, x, edge_index, edge_weight):
    n = x.shape[0]
    fp = iw.shape[-1]
    cp = lin_w.shape[1]

    a_norm = _build_adj(edge_index, edge_weight, n)

    xp = jnp.zeros((n, fp), jnp.float32).at[:, :x.shape[1]].set(x)
    xp = xp.astype(jnp.bfloat16)

    w_state = pl.pallas_call(
        _evolve_kernel,
        out_shape=jax.ShapeDtypeStruct((2, fp, fp), jnp.float32),
        grid=(2,),
        in_specs=[
            pl.BlockSpec((1, fp, fp), lambda l: (l, 0, 0)),
            pl.BlockSpec((1, fp, 6 * fp), lambda l: (l, 0, 0)),
            pl.BlockSpec((1, 1, 6 * fp), lambda l: (l, 0, 0)),
        ],
        out_specs=pl.BlockSpec((1, fp, fp), lambda l: (l, 0, 0)),
        compiler_params=pltpu.CompilerParams(
            dimension_semantics=("parallel",)),
    )(iw, w_cat, b_cat).astype(jnp.bfloat16)

    grid = n // TILE
    y = pl.pallas_call(
        _layer1_kernel,
        out_shape=jax.ShapeDtypeStruct((n, fp), jnp.bfloat16),
        grid=(grid,),
        in_specs=[
            pl.BlockSpec((TILE, n), lambda i: (i, 0)),
            pl.BlockSpec((n, fp), lambda i: (0, 0)),
            pl.BlockSpec((2, fp, fp), lambda i: (0, 0, 0)),
        ],
        out_specs=pl.BlockSpec((TILE, fp), lambda i: (i, 0)),
        compiler_params=pltpu.CompilerParams(
            dimension_semantics=("parallel",),
            vmem_limit_bytes=48 * 1024 * 1024),
    )(a_norm, xp, w_state)

    out = pl.pallas_call(
        _head_kernel,
        out_shape=jax.ShapeDtypeStruct((n, cp), jnp.float32),
        grid=(grid,),
        in_specs=[
            pl.BlockSpec((TILE, n), lambda i: (i, 0)),
            pl.BlockSpec((n, fp), lambda i: (0, 0)),
            pl.BlockSpec((fp, cp), lambda i: (0, 0)),
            pl.BlockSpec((1, cp), lambda i: (0, 0)),
        ],
        out_specs=pl.BlockSpec((TILE, cp), lambda i: (i, 0)),
        compiler_params=pltpu.CompilerParams(
            dimension_semantics=("parallel",),
            vmem_limit_bytes=48 * 1024 * 1024),
    )(a_norm, y, lin_w, lin_b)

    return out[:, :NCLS]


# speedup vs baseline: 4.2132x; 4.1959x over previous
import jax
import jax.numpy as jnp
from jax.experimental import pallas as pl
from jax.experimental.pallas import tpu as pltpu

FP = 128
CP = 128
NCLS = 40
TILE = 256


def _evolve_kernel(iw_ref, wcat_ref, bcat_ref, o_ref):
    h = iw_ref[0]
    fp = h.shape[-1]
    g = jnp.dot(h.astype(jnp.bfloat16), wcat_ref[0],
                preferred_element_type=jnp.float32) + bcat_ref[0]
    r = jax.nn.sigmoid(g[:, :fp] + g[:, 3 * fp:4 * fp])
    z = jax.nn.sigmoid(g[:, fp:2 * fp] + g[:, 4 * fp:5 * fp])
    n = jnp.tanh(g[:, 2 * fp:3 * fp] + r * g[:, 5 * fp:6 * fp])
    o_ref[0] = (1.0 - z) * n + z * h


def _layer1_kernel(a_ref, xs_ref, dr_ref, w_ref, o_ref):
    a = a_ref[...].astype(jnp.bfloat16)
    ax = jnp.dot(a, xs_ref[...], preferred_element_type=jnp.float32)
    ax = ax * dr_ref[0, 0][:, None]
    h = jnp.maximum(
        jnp.dot(ax.astype(jnp.bfloat16), w_ref[0],
                preferred_element_type=jnp.float32), 0.0)
    y = jnp.dot(h.astype(jnp.bfloat16), w_ref[1],
                preferred_element_type=jnp.float32)
    o_ref[...] = (y * dr_ref[0, 0][:, None]).astype(o_ref.dtype)


def _head_kernel(a_ref, ys_ref, dr_ref, lw_ref, lb_ref, o_ref):
    a = a_ref[...].astype(jnp.bfloat16)
    ay = jnp.dot(a, ys_ref[...], preferred_element_type=jnp.float32)
    h = jnp.maximum(ay * dr_ref[0, 0][:, None], 0.0)
    logits = jnp.dot(h.astype(jnp.bfloat16), lw_ref[...],
                     preferred_element_type=jnp.float32) + lb_ref[...]
    m = jnp.max(logits, axis=-1, keepdims=True)
    s = logits - m
    o_ref[...] = s - jnp.log(jnp.sum(jnp.exp(s), axis=-1, keepdims=True))


def _build_raw_adj(edge_index, edge_weight, n):
    src, dst = edge_index[0], edge_index[1]
    has_loop = jnp.zeros((n,), jnp.int32).at[src].max(
        (src == dst).astype(jnp.int32))
    loop_w = jnp.where(has_loop > 0, 0.0, 1.0).astype(edge_weight.dtype)
    deg = loop_w + jnp.zeros((n,), edge_weight.dtype).at[dst].add(edge_weight)
    dinv = jnp.where(deg > 0, jax.lax.rsqrt(deg), 0.0)
    a = jnp.zeros((n, n), edge_weight.dtype).at[dst, src].add(edge_weight)
    di = jnp.arange(n, dtype=jnp.int32)
    a = a.at[di, di].add(loop_w)
    return a, dinv


def kernel(iw, w_cat, b_cat, lin_w, lin_b, x, edge_index, edge_weight):
    n = x.shape[0]
    fp = iw.shape[-1]
    cp = lin_w.shape[1]

    a_raw, dinv = _build_raw_adj(edge_index, edge_weight, n)
    dinv_rows = dinv.reshape(n // TILE, 1, TILE)

    xp = jnp.zeros((n, fp), jnp.float32).at[:, :x.shape[1]].set(x)
    xs = (xp * dinv[:, None]).astype(jnp.bfloat16)

    w_state = pl.pallas_call(
        _evolve_kernel,
        out_shape=jax.ShapeDtypeStruct((2, fp, fp), jnp.float32),
        grid=(2,),
        in_specs=[
            pl.BlockSpec((1, fp, fp), lambda l: (l, 0, 0)),
            pl.BlockSpec((1, fp, 6 * fp), lambda l: (l, 0, 0)),
            pl.BlockSpec((1, 1, 6 * fp), lambda l: (l, 0, 0)),
        ],
        out_specs=pl.BlockSpec((1, fp, fp), lambda l: (l, 0, 0)),
        compiler_params=pltpu.CompilerParams(
            dimension_semantics=("parallel",)),
    )(iw, w_cat, b_cat).astype(jnp.bfloat16)

    grid = n // TILE
    ys = pl.pallas_call(
        _layer1_kernel,
        out_shape=jax.ShapeDtypeStruct((n, fp), jnp.bfloat16),
        grid=(grid,),
        in_specs=[
            pl.BlockSpec((TILE, n), lambda i: (i, 0)),
            pl.BlockSpec((n, fp), lambda i: (0, 0)),
            pl.BlockSpec((1, 1, TILE), lambda i: (i, 0, 0)),
            pl.BlockSpec((2, fp, fp), lambda i: (0, 0, 0)),
        ],
        out_specs=pl.BlockSpec((TILE, fp), lambda i: (i, 0)),
        compiler_params=pltpu.CompilerParams(
            dimension_semantics=("parallel",),
            vmem_limit_bytes=48 * 1024 * 1024),
    )(a_raw, xs, dinv_rows, w_state)

    out = pl.pallas_call(
        _head_kernel,
        out_shape=jax.ShapeDtypeStruct((n, cp), jnp.float32),
        grid=(grid,),
        in_specs=[
            pl.BlockSpec((TILE, n), lambda i: (i, 0)),
            pl.BlockSpec((n, fp), lambda i: (0, 0)),
            pl.BlockSpec((1, 1, TILE), lambda i: (i, 0, 0)),
            pl.BlockSpec((fp, cp), lambda i: (0, 0)),
            pl.BlockSpec((1, cp), lambda i: (0, 0)),
        ],
        out_specs=pl.BlockSpec((TILE, cp), lambda i: (i, 0)),
        compiler_params=pltpu.CompilerParams(
            dimension_semantics=("parallel",),
            vmem_limit_bytes=48 * 1024 * 1024),
    )(a_raw, ys, dinv_rows, lin_w, lin_b)

    return out[:, :NCLS]


# speedup vs baseline: 7.4637x; 1.7715x over previous
import jax
import jax.numpy as jnp
from jax.experimental import pallas as pl
from jax.experimental.pallas import tpu as pltpu

FP = 128
CP = 128
NCLS = 40
TILE = 256


def _evolve_kernel(iw_ref, wcat_ref, bcat_ref, o_ref):
    h = iw_ref[0]
    fp = h.shape[-1]
    g = jnp.dot(h.astype(jnp.bfloat16), wcat_ref[0],
                preferred_element_type=jnp.float32) + bcat_ref[0]
    r = jax.nn.sigmoid(g[:, :fp] + g[:, 3 * fp:4 * fp])
    z = jax.nn.sigmoid(g[:, fp:2 * fp] + g[:, 4 * fp:5 * fp])
    n = jnp.tanh(g[:, 2 * fp:3 * fp] + r * g[:, 5 * fp:6 * fp])
    o_ref[0] = (1.0 - z) * n + z * h


def _stats_kernel(a_ref, adiag_ref, ab_ref, deg_ref, diag_ref):
    a = a_ref[...]
    ab_ref[...] = a.astype(jnp.bfloat16)
    deg_ref[0, 0] = jnp.sum(a, axis=1)
    blk = adiag_ref[...]
    t = blk.shape[0]
    eye = (jax.lax.broadcasted_iota(jnp.int32, (t, t), 0) ==
           jax.lax.broadcasted_iota(jnp.int32, (t, t), 1))
    diag_ref[0, 0] = jnp.sum(jnp.where(eye, blk, 0.0), axis=1)


def _layer1_kernel(a_ref, xs_ref, xst_ref, dr_ref, lw_ref, w_ref, o_ref):
    ax = jnp.dot(a_ref[...], xs_ref[...], preferred_element_type=jnp.float32)
    ax = ax + lw_ref[0, 0][:, None] * xst_ref[...].astype(jnp.float32)
    ax = ax * dr_ref[0, 0][:, None]
    h = jnp.maximum(
        jnp.dot(ax.astype(jnp.bfloat16), w_ref[0],
                preferred_element_type=jnp.float32), 0.0)
    y = jnp.dot(h.astype(jnp.bfloat16), w_ref[1],
                preferred_element_type=jnp.float32)
    o_ref[...] = (y * dr_ref[0, 0][:, None]).astype(o_ref.dtype)


def _head_kernel(a_ref, ys_ref, yst_ref, dr_ref, lw_ref, linw_ref, linb_ref,
                 o_ref):
    ay = jnp.dot(a_ref[...], ys_ref[...], preferred_element_type=jnp.float32)
    ay = ay + lw_ref[0, 0][:, None] * yst_ref[...].astype(jnp.float32)
    h = jnp.maximum(ay * dr_ref[0, 0][:, None], 0.0)
    logits = jnp.dot(h.astype(jnp.bfloat16), linw_ref[...],
                     preferred_element_type=jnp.float32) + linb_ref[...]
    m = jnp.max(logits, axis=-1, keepdims=True)
    s = logits - m
    o_ref[...] = s - jnp.log(jnp.sum(jnp.exp(s), axis=-1, keepdims=True))


def kernel(iw, w_cat, b_cat, lin_w, lin_b, x, edge_index, edge_weight):
    n = x.shape[0]
    fp = iw.shape[-1]
    cp = lin_w.shape[1]
    grid = n // TILE

    src, dst = edge_index[0], edge_index[1]
    a_raw = jnp.zeros((n, n), edge_weight.dtype).at[dst, src].add(edge_weight)

    a_bf16, deg_e, diag = pl.pallas_call(
        _stats_kernel,
        out_shape=(
            jax.ShapeDtypeStruct((n, n), jnp.bfloat16),
            jax.ShapeDtypeStruct((grid, 1, TILE), jnp.float32),
            jax.ShapeDtypeStruct((grid, 1, TILE), jnp.float32),
        ),
        grid=(grid,),
        in_specs=[
            pl.BlockSpec((TILE, n), lambda i: (i, 0)),
            pl.BlockSpec((TILE, TILE), lambda i: (i, i)),
        ],
        out_specs=(
            pl.BlockSpec((TILE, n), lambda i: (i, 0)),
            pl.BlockSpec((1, 1, TILE), lambda i: (i, 0, 0)),
            pl.BlockSpec((1, 1, TILE), lambda i: (i, 0, 0)),
        ),
        compiler_params=pltpu.CompilerParams(
            dimension_semantics=("parallel",),
            vmem_limit_bytes=48 * 1024 * 1024),
    )(a_raw, a_raw)

    loop_w = jnp.where(diag.reshape(n) > 0, 0.0, 1.0)
    deg = deg_e.reshape(n) + loop_w
    dinv = jnp.where(deg > 0, jax.lax.rsqrt(deg), 0.0)
    dinv_rows = dinv.reshape(grid, 1, TILE)
    loop_rows = loop_w.reshape(grid, 1, TILE)

    xp = jnp.zeros((n, fp), jnp.float32).at[:, :x.shape[1]].set(x)
    xs = (xp * dinv[:, None]).astype(jnp.bfloat16)

    w_state = pl.pallas_call(
        _evolve_kernel,
        out_shape=jax.ShapeDtypeStruct((2, fp, fp), jnp.float32),
        grid=(2,),
        in_specs=[
            pl.BlockSpec((1, fp, fp), lambda l: (l, 0, 0)),
            pl.BlockSpec((1, fp, 6 * fp), lambda l: (l, 0, 0)),
            pl.BlockSpec((1, 1, 6 * fp), lambda l: (l, 0, 0)),
        ],
        out_specs=pl.BlockSpec((1, fp, fp), lambda l: (l, 0, 0)),
        compiler_params=pltpu.CompilerParams(
            dimension_semantics=("parallel",)),
    )(iw, w_cat, b_cat).astype(jnp.bfloat16)

    ys = pl.pallas_call(
        _layer1_kernel,
        out_shape=jax.ShapeDtypeStruct((n, fp), jnp.bfloat16),
        grid=(grid,),
        in_specs=[
            pl.BlockSpec((TILE, n), lambda i: (i, 0)),
            pl.BlockSpec((n, fp), lambda i: (0, 0)),
            pl.BlockSpec((TILE, fp), lambda i: (i, 0)),
            pl.BlockSpec((1, 1, TILE), lambda i: (i, 0, 0)),
            pl.BlockSpec((1, 1, TILE), lambda i: (i, 0, 0)),
            pl.BlockSpec((2, fp, fp), lambda i: (0, 0, 0)),
        ],
        out_specs=pl.BlockSpec((TILE, fp), lambda i: (i, 0)),
        compiler_params=pltpu.CompilerParams(
            dimension_semantics=("parallel",),
            vmem_limit_bytes=48 * 1024 * 1024),
    )(a_bf16, xs, xs, dinv_rows, loop_rows, w_state)

    out = pl.pallas_call(
        _head_kernel,
        out_shape=jax.ShapeDtypeStruct((n, cp), jnp.float32),
        grid=(grid,),
        in_specs=[
            pl.BlockSpec((TILE, n), lambda i: (i, 0)),
            pl.BlockSpec((n, fp), lambda i: (0, 0)),
            pl.BlockSpec((TILE, fp), lambda i: (i, 0)),
            pl.BlockSpec((1, 1, TILE), lambda i: (i, 0, 0)),
            pl.BlockSpec((1, 1, TILE), lambda i: (i, 0, 0)),
            pl.BlockSpec((fp, cp), lambda i: (0, 0)),
            pl.BlockSpec((1, cp), lambda i: (0, 0)),
        ],
        out_specs=pl.BlockSpec((TILE, cp), lambda i: (i, 0)),
        compiler_params=pltpu.CompilerParams(
            dimension_semantics=("parallel",),
            vmem_limit_bytes=48 * 1024 * 1024),
    )(a_bf16, ys, ys, dinv_rows, loop_rows, lin_w, lin_b)

    return out[:, :NCLS]
